# speedup vs baseline: 1.7069x; 1.7069x over previous
import functools

import jax
import jax.numpy as jnp
from jax.experimental import pallas as pl
from jax.experimental.pallas import tpu as pltpu

E = 8
TOPK = 2
D_MODEL = 768
D_FF = 768
T = 2048
BLK = 128
TILE_T = 256
N_TT = T // TILE_T
KB13 = D_MODEL // BLK
KB2 = D_FF // BLK


def _moe_body(ids_ref, tw_ref, x_ref, w13_ref, s13_ref, w2_ref, s2_ref,
              out_ref, w13d_ref, w2d_ref):
    e = pl.program_id(0)
    ti = pl.program_id(1)

    @pl.when(ti == 0)
    def _dequant():
        for kb in range(KB13):
            sl = pl.ds(kb * BLK, BLK)
            w13d_ref[:, sl] = w13_ref[0, :, sl] * s13_ref[0, kb, :][:, None]
        for kb in range(KB2):
            sl = pl.ds(kb * BLK, BLK)
            w2d_ref[:, sl] = w2_ref[0, :, sl] * s2_ref[0, kb, :][:, None]

    x = x_ref[...]
    h = jax.lax.dot_general(x, w13d_ref[...], (((1,), (1,)), ((), ())),
                            preferred_element_type=jnp.float32)
    gate = h[:, :D_FF]
    up = h[:, D_FF:]
    act = gate / (1.0 + jnp.exp(-gate)) * up
    y = jax.lax.dot_general(act, w2d_ref[...], (((1,), (1,)), ((), ())),
                            preferred_element_type=jnp.float32)

    ids = ids_ref[...]
    tw = tw_ref[...]
    cw = jnp.sum(jnp.where(ids == e, tw, 0.0), axis=1, keepdims=True)
    contrib = y * cw

    row = pl.ds(ti * TILE_T, TILE_T)

    @pl.when(e == 0)
    def _init():
        out_ref[row, :] = contrib

    @pl.when(e != 0)
    def _acc():
        out_ref[row, :] = out_ref[row, :] + contrib


@jax.jit
def _moe_dense(x, topk_ids, topk_weights, w13_fp8, s13e, w2_fp8, s2e):
    grid = (E, N_TT)
    return pl.pallas_call(
        _moe_body,
        grid=grid,
        in_specs=[
            pl.BlockSpec((TILE_T, TOPK), lambda e, ti: (ti, 0)),
            pl.BlockSpec((TILE_T, TOPK), lambda e, ti: (ti, 0)),
            pl.BlockSpec((TILE_T, D_MODEL), lambda e, ti: (ti, 0)),
            pl.BlockSpec((1, 2 * D_FF, D_MODEL), lambda e, ti: (e, 0, 0)),
            pl.BlockSpec((1, KB13, 2 * D_FF), lambda e, ti: (e, 0, 0)),
            pl.BlockSpec((1, D_MODEL, D_FF), lambda e, ti: (e, 0, 0)),
            pl.BlockSpec((1, KB2, D_MODEL), lambda e, ti: (e, 0, 0)),
        ],
        out_specs=pl.BlockSpec((T, D_MODEL), lambda e, ti: (0, 0)),
        out_shape=jax.ShapeDtypeStruct((T, D_MODEL), jnp.float32),
        scratch_shapes=[
            pltpu.VMEM((2 * D_FF, D_MODEL), jnp.float32),
            pltpu.VMEM((D_MODEL, D_FF), jnp.float32),
        ],
    )(topk_ids, topk_weights, x, w13_fp8, s13e, w2_fp8, s2e)


def kernel(x, topk_ids, topk_weights, moe_n_slice, n_expert_slice, ep_shift,
           w13_fp8, w13_scale_inv, w2_fp8, w2_scale_inv):
    s13e = jnp.repeat(w13_scale_inv.transpose(0, 2, 1), BLK, axis=2)
    s2e = jnp.repeat(w2_scale_inv.transpose(0, 2, 1), BLK, axis=2)
    return _moe_dense(x, topk_ids.astype(jnp.int32), topk_weights,
                      w13_fp8, s13e, w2_fp8, s2e)
